# baseline (speedup 1.0000x reference)
import functools
import math

import jax
import jax.numpy as jnp
from jax import lax
from jax.experimental import pallas as pl
from jax.experimental.pallas import tpu as pltpu
from jax.experimental.pallas import tpu_sc as plsc

D_MODEL = 1024
SCALE = math.sqrt(float(D_MODEL))
LANES = 16
VECS = D_MODEL // LANES

NC = 2
NS = 16
NW = NC * NS

B_TOT = 4 * 4096
B_PER_W = B_TOT // NW

ARENA = 120
SLOT_OFF = (0, 32, 64, 96)
SLOT_ROWS = (32, 32, 32, 24)
NSLOT = 4
CHUNKS = []
_off = 0
_k = 0
while _off < B_PER_W:
    _s = _k % NSLOT
    _n = min(SLOT_ROWS[_s], B_PER_W - _off)
    CHUNKS.append((_off, _n, _s))
    _off += _n
    _k += 1
NCHUNK = len(CHUNKS)

_mesh = plsc.VectorSubcoreMesh(core_axis_name="c", subcore_axis_name="s")


@functools.partial(
    pl.kernel,
    mesh=_mesh,
    out_type=jax.ShapeDtypeStruct((B_TOT, D_MODEL), jnp.float32),
    scratch_types=[
        pltpu.VMEM((B_PER_W,), jnp.int32),
        pltpu.VMEM((ARENA, D_MODEL), jnp.float32),
        pltpu.SemaphoreType.DMA,
        pltpu.SemaphoreType.DMA,
        pltpu.SemaphoreType.DMA,
        pltpu.SemaphoreType.DMA,
        pltpu.SemaphoreType.DMA,
        pltpu.SemaphoreType.DMA,
        pltpu.SemaphoreType.DMA,
        pltpu.SemaphoreType.DMA,
    ],
)
def _emb_lookup(x_hbm, lut_hbm, out_hbm, idx_v, arena,
                si0, si1, si2, si3, so0, so1, so2, so3):
    wid = lax.axis_index("s") * NC + lax.axis_index("c")
    base = wid * B_PER_W
    pltpu.sync_copy(x_hbm.at[pl.ds(base, B_PER_W)], idx_v)
    scale = jnp.full((LANES,), SCALE, jnp.float32)

    sin = [si0, si1, si2, si3]
    sout = [so0, so1, so2, so3]

    def slot(s, n):
        return arena.at[pl.ds(SLOT_OFF[s], n)]

    def gather(k):
        off, n, s = CHUNKS[k]
        return pltpu.async_copy(
            lut_hbm.at[idx_v.at[pl.ds(off, n)]], slot(s, n), sin[s])

    def outcopy(k):
        off, n, s = CHUNKS[k]
        return pltpu.async_copy(
            slot(s, n), out_hbm.at[pl.ds(base + off, n)], sout[s])

    def multiply(k):
        _, n, s = CHUNKS[k]
        buf = slot(s, n)

        def mul_row(r, _):
            for j in range(VECS):
                sl = pl.ds(j * LANES, LANES)
                buf[r, sl] = buf[r, sl] * scale
            return 0

        lax.fori_loop(0, n, mul_row, 0)

    copies_in = {0: gather(0), 1: gather(1), 2: gather(2)}
    copies_out = {}
    for k in range(NCHUNK):
        copies_in[k].wait()
        multiply(k)
        copies_out[k] = outcopy(k)
        if k + 3 < NCHUNK:
            if k - 1 >= 0:
                copies_out[k - 1].wait()
            copies_in[k + 3] = gather(k + 3)
    for k in range(max(0, NCHUNK - 4), NCHUNK):
        copies_out[k].wait()


def kernel(x, lut):
    xf = x.reshape(B_TOT)
    out = _emb_lookup(xf, lut)
    return out.reshape(4, 4096, D_MODEL)
